# baseline (speedup 1.0000x reference)
import functools

import jax
import jax.numpy as jnp
from jax import lax
from jax.experimental import pallas as pl
from jax.experimental.pallas import tpu as pltpu
from jax.experimental.pallas import tpu_sc as plsc

_NC = 2
_NS = 16
_L = 16


def _build(B, F, V, D):
    NW = _NC * _NS
    BPW = B // NW
    C = 64
    NCH = BPW // C
    IPC = C * F
    NSUB = IPC // 128
    assert B % NW == 0 and BPW % C == 0 and IPC % 128 == 0

    mesh = plsc.VectorSubcoreMesh(core_axis_name="c", subcore_axis_name="s")

    @functools.partial(
        pl.kernel,
        mesh=mesh,
        compiler_params=pltpu.CompilerParams(
            needs_layout_passes=False, use_tc_tiling_on_sc=False),
        out_type=jax.ShapeDtypeStruct((B,), jnp.float32),
        scratch_types=[
            pltpu.VMEM((IPC,), jnp.int32),
            pltpu.VMEM((NSUB, 128), jnp.int32),
            pltpu.VMEM((NSUB, 128), jnp.int32),
            pltpu.VMEM((IPC, _L), jnp.float32),
            pltpu.VMEM((IPC + _L,), jnp.float32),
            pltpu.VMEM((C,), jnp.float32),
            pltpu.VMEM((_L,), jnp.float32),
            pltpu.SemaphoreType.DMA,
        ],
    )
    def fm(x_hbm, emb_hbm, lin_hbm, lb_hbm, out_hbm,
           x_v, idx_v, slot_v, rows_v, lin_v, out_v, lb_v, sem):
        cid = lax.axis_index("c")
        sid = lax.axis_index("s")
        wid = sid * _NC + cid
        base = wid * BPW

        pltpu.sync_copy(lb_hbm, lb_v.at[pl.ds(0, 1)])
        lb = lb_v[pl.ds(0, _L)][0]

        lane = lax.iota(jnp.int32, _L)
        m_tail = jnp.where(lane < (F - _L), 1.0, 0.0).astype(jnp.float32)
        m_last = lane == (_L - 1)
        zeros_i = jnp.zeros((_L,), jnp.int32)

        def chunk_body(ci, carry):
            cbase = base + ci * C
            pltpu.sync_copy(x_hbm.at[pl.ds(cbase * F, IPC)], x_v)

            def idx_body(r, carry2):
                for j in range(128 // _L):
                    p0 = r * 128 + j * _L
                    pos = p0 + lane
                    fld = lax.rem(pos, F)
                    idx = x_v[pl.ds(p0, _L)] + fld * V
                    idx_v[r, pl.ds(j * _L, _L)] = idx
                    slot_v[r, pl.ds(j * _L, _L)] = (
                        (idx & jnp.int32(~2047))
                        | ((idx & 255) << 3)
                        | ((idx >> 8) & 7))
                return carry2
            lax.fori_loop(0, NSUB, idx_body, 0)

            copies = []
            for j in range(NSUB):
                copies.append(pltpu.async_copy(
                    emb_hbm.at[slot_v.at[j]],
                    rows_v.at[pl.ds(j * 128, 128)], sem))
                copies.append(pltpu.async_copy(
                    lin_hbm.at[idx_v.at[j]],
                    lin_v.at[pl.ds(j * 128, 128)], sem))
            for cp in copies:
                cp.wait()

            def row_body(b, carry2):
                rbase = b * F
                acc = rows_v[rbase, :]
                acc2 = acc * acc
                for f in range(1, F):
                    r = rows_v[rbase + f, :]
                    acc = acc + r
                    acc2 = acc2 + r * r
                lv = lin_v[pl.ds(rbase, _L)] + lin_v[pl.ds(rbase + _L, _L)] * m_tail
                cs = plsc.cumsum(acc)
                cq = plsc.cumsum(acc2)
                cl = plsc.cumsum(lv)
                logit = cl + lb + 0.5 * (cs * cs - cq)
                sig = 1.0 / (1.0 + jnp.exp(-logit))
                plsc.store_scatter(out_v, [zeros_i + b], sig, mask=m_last)
                return carry2
            lax.fori_loop(0, C, row_body, 0)

            pltpu.sync_copy(out_v, out_hbm.at[pl.ds(cbase, C)])
            return carry
        lax.fori_loop(0, NCH, chunk_body, 0)

    return fm


def _build_transpose(FV, D):
    NW = _NC * _NS
    W = 128
    J = 8
    CW = J * W
    NTC = FV // W
    TAIL = FV - NTC * W
    NB = NTC // J
    KMAX = (NB + NW - 1) // NW
    assert NTC % J == 0 and KMAX % 2 == 0 and TAIL * D % W == 0

    mesh = plsc.VectorSubcoreMesh(core_axis_name="c", subcore_axis_name="s")

    @functools.partial(
        pl.kernel,
        mesh=mesh,
        compiler_params=pltpu.CompilerParams(
            needs_layout_passes=False, use_tc_tiling_on_sc=True),
        out_type=jax.ShapeDtypeStruct((FV * D // W, W), jnp.float32),
        scratch_types=[
            pltpu.VMEM((2, D, CW + 1), jnp.float32),
            pltpu.VMEM((2, CW * D // W, W), jnp.float32),
            pltpu.VMEM((TAIL * D // W, W), jnp.float32),
            pltpu.SemaphoreType.DMA,
            pltpu.SemaphoreType.DMA,
            pltpu.SemaphoreType.DMA,
            pltpu.SemaphoreType.DMA,
        ],
    )
    def transpose(src_hbm, tail_hbm, dst_hbm, tv, rv, lv, si0, si1, so0, so1):
        cid = lax.axis_index("c")
        sid = lax.axis_index("s")
        wid = sid * _NC + cid
        sin = (si0, si1)
        sout = (so0, so1)
        lane = lax.iota(jnp.int32, _L)
        RPB = CW * D // W

        def fire_in(m, p):
            @pl.when(m < NB)
            def _():
                pltpu.async_copy(
                    src_hbm.at[:, pl.ds(m * CW, CW)],
                    tv.at[p].at[:, pl.ds(0, CW)], sin[p])

        def wait_in(m, p):
            @pl.when(m < NB)
            def _():
                pltpu.make_async_copy(
                    src_hbm.at[:, pl.ds(m * CW, CW)],
                    tv.at[p].at[:, pl.ds(0, CW)], sin[p]).wait()

        def fire_out(m, p):
            @pl.when(m < NB)
            def _():
                pltpu.async_copy(
                    rv.at[p], dst_hbm.at[pl.ds(m * RPB, RPB)], sout[p])

        def wait_out(m, p):
            @pl.when(m < NB)
            def _():
                pltpu.make_async_copy(
                    rv.at[p], dst_hbm.at[pl.ds(m * RPB, RPB)],
                    sout[p]).wait()

        def trans(m, p):
            @pl.when(m < NB)
            def _():
                def step(i, carry):
                    for k in range(J):
                        r = i * J + k
                        vals = plsc.load_gather(tv.at[p], [lane, r + 0 * lane])
                        rv[p, i, pl.ds(k * D, D)] = vals
                    return carry
                lax.fori_loop(0, RPB, step, 0)

        def blk(k):
            return wid + NW * k

        fire_in(blk(0), 0)

        def pair_body(g, carry):
            k0 = 2 * g
            fire_in(blk(k0 + 1), 1)
            wait_in(blk(k0), 0)

            @pl.when(g > 0)
            def _():
                wait_out(blk(k0 - 2), 0)
            trans(blk(k0), 0)
            fire_out(blk(k0), 0)

            fire_in(blk(k0 + 2), 0)
            wait_in(blk(k0 + 1), 1)

            @pl.when(g > 0)
            def _():
                wait_out(blk(k0 - 1), 1)
            trans(blk(k0 + 1), 1)
            fire_out(blk(k0 + 1), 1)
            return carry
        lax.fori_loop(0, KMAX // 2, pair_body, 0)

        wait_in(blk(KMAX), 0)
        wait_out(blk(KMAX - 2), 0)
        wait_out(blk(KMAX - 1), 1)

        @pl.when(wid == NW - 1)
        def _():
            pltpu.sync_copy(tail_hbm, lv)
            pltpu.sync_copy(
                lv, dst_hbm.at[pl.ds(NTC * W * D // W, TAIL * D // W)])

    return transpose


def _build_tc_transpose(FV, D):
    BN = 2048
    NBLK = (FV + BN - 1) // BN
    SB = BN // 8
    assert BN % 128 == 0

    def body(x_ref, o_ref):
        eye_d = jnp.eye(D, dtype=jnp.float32)
        perm = jnp.eye(8 * D, dtype=jnp.float32).reshape(8, D, 8 * D)
        z = lax.dot_general(
            x_ref[...], eye_d, (((0,), (0,)), ((), ())),
            preferred_element_type=jnp.float32)
        out = None
        for rr in range(8):
            part = lax.dot_general(
                z[rr * SB:(rr + 1) * SB, :], perm[rr],
                (((1,), (0,)), ((), ())),
                preferred_element_type=jnp.float32)
            out = part if out is None else out + part
        o_ref[...] = out

    return pl.pallas_call(
        body,
        grid=(NBLK,),
        in_specs=[pl.BlockSpec((D, BN), lambda i: (0, i))],
        out_specs=pl.BlockSpec((SB, 8 * D), lambda i: (i, 0)),
        out_shape=jax.ShapeDtypeStruct((NBLK * SB, 8 * D), jnp.float32),
    )


def kernel(x, emb_table, lin_w, lin_b):
    B, F = x.shape
    D = emb_table.shape[1]
    V = emb_table.shape[0] // F
    FV = F * V
    w2 = _build_tc_transpose(FV, D)(emb_table.T)
    emb_2d = w2.reshape(w2.shape[0] * w2.shape[1] // D, D)
    fm = _build(B, F, V, D)
    out = fm(x.reshape(B * F).astype(jnp.int32), emb_2d, lin_w, lin_b)
    return out.reshape(B, 1)


# speedup vs baseline: 1.8505x; 1.8505x over previous
import functools

import jax
import jax.numpy as jnp
from jax import lax
from jax.experimental import pallas as pl
from jax.experimental.pallas import tpu as pltpu
from jax.experimental.pallas import tpu_sc as plsc

_NC = 2
_NS = 16
_L = 16


def _build(B, F, V, D):
    NW = _NC * _NS
    BPW = B // NW
    C = 64
    NCH = BPW // C
    IPC = C * F
    NSUB = IPC // 128
    assert B % NW == 0 and BPW % C == 0 and IPC % 128 == 0

    mesh = plsc.VectorSubcoreMesh(core_axis_name="c", subcore_axis_name="s")

    @functools.partial(
        pl.kernel,
        mesh=mesh,
        compiler_params=pltpu.CompilerParams(
            needs_layout_passes=False, use_tc_tiling_on_sc=False),
        out_type=jax.ShapeDtypeStruct((B,), jnp.float32),
        scratch_types=[
            pltpu.VMEM((IPC,), jnp.int32),
            pltpu.VMEM((NSUB, 128), jnp.int32),
            pltpu.VMEM((NSUB, 128), jnp.int32),
            pltpu.VMEM((IPC, _L), jnp.float32),
            pltpu.VMEM((IPC + _L,), jnp.float32),
            pltpu.VMEM((C,), jnp.float32),
            pltpu.VMEM((_L,), jnp.float32),
            pltpu.SemaphoreType.DMA,
        ],
    )
    def fm(x_hbm, emb_hbm, lin_hbm, lb_hbm, out_hbm,
           x_v, idx_v, slot_v, rows_v, lin_v, out_v, lb_v, sem):
        cid = lax.axis_index("c")
        sid = lax.axis_index("s")
        wid = sid * _NC + cid
        base = wid * BPW

        pltpu.sync_copy(lb_hbm, lb_v.at[pl.ds(0, 1)])
        lb = lb_v[pl.ds(0, _L)][0]

        lane = lax.iota(jnp.int32, _L)
        m_tail = jnp.where(lane < (F - _L), 1.0, 0.0).astype(jnp.float32)
        m_last = lane == (_L - 1)
        zeros_i = jnp.zeros((_L,), jnp.int32)

        def chunk_body(ci, carry):
            cbase = base + ci * C
            pltpu.sync_copy(x_hbm.at[pl.ds(cbase * F, IPC)], x_v)

            def idx_body(r, carry2):
                for j in range(128 // _L):
                    p0 = r * 128 + j * _L
                    pos = p0 + lane
                    fld = lax.rem(pos, F)
                    idx = x_v[pl.ds(p0, _L)] + fld * V
                    idx_v[r, pl.ds(j * _L, _L)] = idx
                    slot_v[r, pl.ds(j * _L, _L)] = (
                        (idx & jnp.int32(~8191))
                        | ((idx & 1023) << 3)
                        | ((idx >> 10) & 7))
                return carry2
            lax.fori_loop(0, NSUB, idx_body, 0)

            copies = []
            for j in range(NSUB):
                copies.append(pltpu.async_copy(
                    emb_hbm.at[slot_v.at[j]],
                    rows_v.at[pl.ds(j * 128, 128)], sem))
                copies.append(pltpu.async_copy(
                    lin_hbm.at[idx_v.at[j]],
                    lin_v.at[pl.ds(j * 128, 128)], sem))
            for cp in copies:
                cp.wait()

            def row_body(b, carry2):
                rbase = b * F
                acc = rows_v[rbase, :]
                acc2 = acc * acc
                for f in range(1, F):
                    r = rows_v[rbase + f, :]
                    acc = acc + r
                    acc2 = acc2 + r * r
                lv = lin_v[pl.ds(rbase, _L)] + lin_v[pl.ds(rbase + _L, _L)] * m_tail
                cs = plsc.cumsum(acc)
                cq = plsc.cumsum(acc2)
                cl = plsc.cumsum(lv)
                logit = cl + lb + 0.5 * (cs * cs - cq)
                sig = 1.0 / (1.0 + jnp.exp(-logit))
                plsc.store_scatter(out_v, [zeros_i + b], sig, mask=m_last)
                return carry2
            lax.fori_loop(0, C, row_body, 0)

            pltpu.sync_copy(out_v, out_hbm.at[pl.ds(cbase, C)])
            return carry
        lax.fori_loop(0, NCH, chunk_body, 0)

    return fm


def _build_transpose(FV, D):
    NW = _NC * _NS
    W = 128
    J = 8
    CW = J * W
    NTC = FV // W
    TAIL = FV - NTC * W
    NB = NTC // J
    KMAX = (NB + NW - 1) // NW
    assert NTC % J == 0 and KMAX % 2 == 0 and TAIL * D % W == 0

    mesh = plsc.VectorSubcoreMesh(core_axis_name="c", subcore_axis_name="s")

    @functools.partial(
        pl.kernel,
        mesh=mesh,
        compiler_params=pltpu.CompilerParams(
            needs_layout_passes=False, use_tc_tiling_on_sc=True),
        out_type=jax.ShapeDtypeStruct((FV * D // W, W), jnp.float32),
        scratch_types=[
            pltpu.VMEM((2, D, CW + 1), jnp.float32),
            pltpu.VMEM((2, CW * D // W, W), jnp.float32),
            pltpu.VMEM((TAIL * D // W, W), jnp.float32),
            pltpu.SemaphoreType.DMA,
            pltpu.SemaphoreType.DMA,
            pltpu.SemaphoreType.DMA,
            pltpu.SemaphoreType.DMA,
        ],
    )
    def transpose(src_hbm, tail_hbm, dst_hbm, tv, rv, lv, si0, si1, so0, so1):
        cid = lax.axis_index("c")
        sid = lax.axis_index("s")
        wid = sid * _NC + cid
        sin = (si0, si1)
        sout = (so0, so1)
        lane = lax.iota(jnp.int32, _L)
        RPB = CW * D // W

        def fire_in(m, p):
            @pl.when(m < NB)
            def _():
                pltpu.async_copy(
                    src_hbm.at[:, pl.ds(m * CW, CW)],
                    tv.at[p].at[:, pl.ds(0, CW)], sin[p])

        def wait_in(m, p):
            @pl.when(m < NB)
            def _():
                pltpu.make_async_copy(
                    src_hbm.at[:, pl.ds(m * CW, CW)],
                    tv.at[p].at[:, pl.ds(0, CW)], sin[p]).wait()

        def fire_out(m, p):
            @pl.when(m < NB)
            def _():
                pltpu.async_copy(
                    rv.at[p], dst_hbm.at[pl.ds(m * RPB, RPB)], sout[p])

        def wait_out(m, p):
            @pl.when(m < NB)
            def _():
                pltpu.make_async_copy(
                    rv.at[p], dst_hbm.at[pl.ds(m * RPB, RPB)],
                    sout[p]).wait()

        def trans(m, p):
            @pl.when(m < NB)
            def _():
                def step(i, carry):
                    for k in range(J):
                        r = i * J + k
                        vals = plsc.load_gather(tv.at[p], [lane, r + 0 * lane])
                        rv[p, i, pl.ds(k * D, D)] = vals
                    return carry
                lax.fori_loop(0, RPB, step, 0)

        def blk(k):
            return wid + NW * k

        fire_in(blk(0), 0)

        def pair_body(g, carry):
            k0 = 2 * g
            fire_in(blk(k0 + 1), 1)
            wait_in(blk(k0), 0)

            @pl.when(g > 0)
            def _():
                wait_out(blk(k0 - 2), 0)
            trans(blk(k0), 0)
            fire_out(blk(k0), 0)

            fire_in(blk(k0 + 2), 0)
            wait_in(blk(k0 + 1), 1)

            @pl.when(g > 0)
            def _():
                wait_out(blk(k0 - 1), 1)
            trans(blk(k0 + 1), 1)
            fire_out(blk(k0 + 1), 1)
            return carry
        lax.fori_loop(0, KMAX // 2, pair_body, 0)

        wait_in(blk(KMAX), 0)
        wait_out(blk(KMAX - 2), 0)
        wait_out(blk(KMAX - 1), 1)

        @pl.when(wid == NW - 1)
        def _():
            pltpu.sync_copy(tail_hbm, lv)
            pltpu.sync_copy(
                lv, dst_hbm.at[pl.ds(NTC * W * D // W, TAIL * D // W)])

    return transpose


def _build_tc_transpose(FV, D):
    BN = 8192
    NBLK = (FV + BN - 1) // BN
    SB = BN // 8
    assert BN % 128 == 0

    def body(x_ref, o_ref):
        eye_d = jnp.eye(D, dtype=jnp.float32)
        perm = jnp.eye(8 * D, dtype=jnp.float32).reshape(8, D, 8 * D)
        z = lax.dot_general(
            x_ref[...], eye_d, (((0,), (0,)), ((), ())),
            preferred_element_type=jnp.float32)
        out = None
        for rr in range(8):
            part = lax.dot_general(
                z[rr * SB:(rr + 1) * SB, :], perm[rr],
                (((1,), (0,)), ((), ())),
                preferred_element_type=jnp.float32)
            out = part if out is None else out + part
        o_ref[...] = out

    return pl.pallas_call(
        body,
        grid=(NBLK,),
        in_specs=[pl.BlockSpec((D, BN), lambda i: (0, i))],
        out_specs=pl.BlockSpec((SB, 8 * D), lambda i: (i, 0)),
        out_shape=jax.ShapeDtypeStruct((NBLK * SB, 8 * D), jnp.float32),
    )


def kernel(x, emb_table, lin_w, lin_b):
    B, F = x.shape
    D = emb_table.shape[1]
    V = emb_table.shape[0] // F
    FV = F * V
    w2 = _build_tc_transpose(FV, D)(emb_table.T)
    emb_2d = w2.reshape(w2.shape[0] * w2.shape[1] // D, D)
    fm = _build(B, F, V, D)
    out = fm(x.reshape(B * F).astype(jnp.int32), emb_2d, lin_w, lin_b)
    return out.reshape(B, 1)


# speedup vs baseline: 2.1158x; 1.1434x over previous
import functools

import jax
import jax.numpy as jnp
from jax import lax
from jax.experimental import pallas as pl
from jax.experimental.pallas import tpu as pltpu
from jax.experimental.pallas import tpu_sc as plsc

_NC = 2
_NS = 16
_L = 16


def _build(B, F, V, D):
    NW = _NC * _NS
    BPW = B // NW
    C = 64
    NCH = BPW // C
    IPC = C * F
    NSUB = IPC // 128
    assert B % NW == 0 and BPW % C == 0 and IPC % 128 == 0

    mesh = plsc.VectorSubcoreMesh(core_axis_name="c", subcore_axis_name="s")

    @functools.partial(
        pl.kernel,
        mesh=mesh,
        compiler_params=pltpu.CompilerParams(
            needs_layout_passes=False, use_tc_tiling_on_sc=False),
        out_type=jax.ShapeDtypeStruct((B,), jnp.float32),
        scratch_types=[
            pltpu.VMEM((IPC,), jnp.int32),
            pltpu.VMEM((NSUB, 128), jnp.int32),
            pltpu.VMEM((NSUB, 128), jnp.int32),
            pltpu.VMEM((IPC, _L), jnp.float32),
            pltpu.VMEM((IPC + _L,), jnp.float32),
            pltpu.VMEM((C,), jnp.float32),
            pltpu.VMEM((_L,), jnp.float32),
            pltpu.SemaphoreType.DMA,
        ],
    )
    def fm(x_hbm, emb_hbm, lin_hbm, lb_hbm, out_hbm,
           x_v, idx_v, slot_v, rows_v, lin_v, out_v, lb_v, sem):
        cid = lax.axis_index("c")
        sid = lax.axis_index("s")
        wid = sid * _NC + cid
        base = wid * BPW

        pltpu.sync_copy(lb_hbm, lb_v.at[pl.ds(0, 1)])
        lb = lb_v[pl.ds(0, _L)][0]

        lane = lax.iota(jnp.int32, _L)
        m_tail = jnp.where(lane < (F - _L), 1.0, 0.0).astype(jnp.float32)
        m_last = lane == (_L - 1)
        zeros_i = jnp.zeros((_L,), jnp.int32)

        def chunk_body(ci, carry):
            cbase = base + ci * C
            pltpu.sync_copy(x_hbm.at[pl.ds(cbase * F, IPC)], x_v)

            def idx_body(r, carry2):
                for j in range(128 // _L):
                    p0 = r * 128 + j * _L
                    pos = p0 + lane
                    fld = lax.rem(pos, F)
                    idx = x_v[pl.ds(p0, _L)] + fld * V
                    idx_v[r, pl.ds(j * _L, _L)] = idx
                    slot_v[r, pl.ds(j * _L, _L)] = (
                        (idx & jnp.int32(~32767))
                        | ((idx & 4095) << 3)
                        | ((idx >> 12) & 7))
                return carry2
            lax.fori_loop(0, NSUB, idx_body, 0)

            copies = []
            for j in range(NSUB):
                copies.append(pltpu.async_copy(
                    emb_hbm.at[slot_v.at[j]],
                    rows_v.at[pl.ds(j * 128, 128)], sem))
                copies.append(pltpu.async_copy(
                    lin_hbm.at[idx_v.at[j]],
                    lin_v.at[pl.ds(j * 128, 128)], sem))
            for cp in copies:
                cp.wait()

            def row_body(b, carry2):
                rbase = b * F
                acc = rows_v[rbase, :]
                acc2 = acc * acc
                for f in range(1, F):
                    r = rows_v[rbase + f, :]
                    acc = acc + r
                    acc2 = acc2 + r * r
                lv = lin_v[pl.ds(rbase, _L)] + lin_v[pl.ds(rbase + _L, _L)] * m_tail
                cs = plsc.cumsum(acc)
                cq = plsc.cumsum(acc2)
                cl = plsc.cumsum(lv)
                logit = cl + lb + 0.5 * (cs * cs - cq)
                sig = 1.0 / (1.0 + jnp.exp(-logit))
                plsc.store_scatter(out_v, [zeros_i + b], sig, mask=m_last)
                return carry2
            lax.fori_loop(0, C, row_body, 0)

            pltpu.sync_copy(out_v, out_hbm.at[pl.ds(cbase, C)])
            return carry
        lax.fori_loop(0, NCH, chunk_body, 0)

    return fm


def _build_transpose(FV, D):
    NW = _NC * _NS
    W = 128
    J = 8
    CW = J * W
    NTC = FV // W
    TAIL = FV - NTC * W
    NB = NTC // J
    KMAX = (NB + NW - 1) // NW
    assert NTC % J == 0 and KMAX % 2 == 0 and TAIL * D % W == 0

    mesh = plsc.VectorSubcoreMesh(core_axis_name="c", subcore_axis_name="s")

    @functools.partial(
        pl.kernel,
        mesh=mesh,
        compiler_params=pltpu.CompilerParams(
            needs_layout_passes=False, use_tc_tiling_on_sc=True),
        out_type=jax.ShapeDtypeStruct((FV * D // W, W), jnp.float32),
        scratch_types=[
            pltpu.VMEM((2, D, CW + 1), jnp.float32),
            pltpu.VMEM((2, CW * D // W, W), jnp.float32),
            pltpu.VMEM((TAIL * D // W, W), jnp.float32),
            pltpu.SemaphoreType.DMA,
            pltpu.SemaphoreType.DMA,
            pltpu.SemaphoreType.DMA,
            pltpu.SemaphoreType.DMA,
        ],
    )
    def transpose(src_hbm, tail_hbm, dst_hbm, tv, rv, lv, si0, si1, so0, so1):
        cid = lax.axis_index("c")
        sid = lax.axis_index("s")
        wid = sid * _NC + cid
        sin = (si0, si1)
        sout = (so0, so1)
        lane = lax.iota(jnp.int32, _L)
        RPB = CW * D // W

        def fire_in(m, p):
            @pl.when(m < NB)
            def _():
                pltpu.async_copy(
                    src_hbm.at[:, pl.ds(m * CW, CW)],
                    tv.at[p].at[:, pl.ds(0, CW)], sin[p])

        def wait_in(m, p):
            @pl.when(m < NB)
            def _():
                pltpu.make_async_copy(
                    src_hbm.at[:, pl.ds(m * CW, CW)],
                    tv.at[p].at[:, pl.ds(0, CW)], sin[p]).wait()

        def fire_out(m, p):
            @pl.when(m < NB)
            def _():
                pltpu.async_copy(
                    rv.at[p], dst_hbm.at[pl.ds(m * RPB, RPB)], sout[p])

        def wait_out(m, p):
            @pl.when(m < NB)
            def _():
                pltpu.make_async_copy(
                    rv.at[p], dst_hbm.at[pl.ds(m * RPB, RPB)],
                    sout[p]).wait()

        def trans(m, p):
            @pl.when(m < NB)
            def _():
                def step(i, carry):
                    for k in range(J):
                        r = i * J + k
                        vals = plsc.load_gather(tv.at[p], [lane, r + 0 * lane])
                        rv[p, i, pl.ds(k * D, D)] = vals
                    return carry
                lax.fori_loop(0, RPB, step, 0)

        def blk(k):
            return wid + NW * k

        fire_in(blk(0), 0)

        def pair_body(g, carry):
            k0 = 2 * g
            fire_in(blk(k0 + 1), 1)
            wait_in(blk(k0), 0)

            @pl.when(g > 0)
            def _():
                wait_out(blk(k0 - 2), 0)
            trans(blk(k0), 0)
            fire_out(blk(k0), 0)

            fire_in(blk(k0 + 2), 0)
            wait_in(blk(k0 + 1), 1)

            @pl.when(g > 0)
            def _():
                wait_out(blk(k0 - 1), 1)
            trans(blk(k0 + 1), 1)
            fire_out(blk(k0 + 1), 1)
            return carry
        lax.fori_loop(0, KMAX // 2, pair_body, 0)

        wait_in(blk(KMAX), 0)
        wait_out(blk(KMAX - 2), 0)
        wait_out(blk(KMAX - 1), 1)

        @pl.when(wid == NW - 1)
        def _():
            pltpu.sync_copy(tail_hbm, lv)
            pltpu.sync_copy(
                lv, dst_hbm.at[pl.ds(NTC * W * D // W, TAIL * D // W)])

    return transpose


def _build_tc_transpose(FV, D):
    BN = 32768
    NBLK = (FV + BN - 1) // BN
    SB = BN // 8
    assert BN % 128 == 0

    def body(x_ref, o_ref):
        eye_d = jnp.eye(D, dtype=jnp.float32)
        perm = jnp.eye(8 * D, dtype=jnp.float32).reshape(8, D, 8 * D)
        z = lax.dot_general(
            x_ref[...], eye_d, (((0,), (0,)), ((), ())),
            preferred_element_type=jnp.float32)
        out = None
        for rr in range(8):
            part = lax.dot_general(
                z[rr * SB:(rr + 1) * SB, :], perm[rr],
                (((1,), (0,)), ((), ())),
                preferred_element_type=jnp.float32)
            out = part if out is None else out + part
        o_ref[...] = out

    return pl.pallas_call(
        body,
        grid=(NBLK,),
        in_specs=[pl.BlockSpec((D, BN), lambda i: (0, i))],
        out_specs=pl.BlockSpec((SB, 8 * D), lambda i: (i, 0)),
        out_shape=jax.ShapeDtypeStruct((NBLK * SB, 8 * D), jnp.float32),
    )


def kernel(x, emb_table, lin_w, lin_b):
    B, F = x.shape
    D = emb_table.shape[1]
    V = emb_table.shape[0] // F
    FV = F * V
    w2 = _build_tc_transpose(FV, D)(emb_table.T)
    emb_2d = w2.reshape(w2.shape[0] * w2.shape[1] // D, D)
    fm = _build(B, F, V, D)
    out = fm(x.reshape(B * F).astype(jnp.int32), emb_2d, lin_w, lin_b)
    return out.reshape(B, 1)


# speedup vs baseline: 2.1902x; 1.0352x over previous
import functools

import jax
import jax.numpy as jnp
from jax import lax
from jax.experimental import pallas as pl
from jax.experimental.pallas import tpu as pltpu
from jax.experimental.pallas import tpu_sc as plsc

_NC = 2
_NS = 16
_L = 16


def _build(B, F, V, D):
    NW = _NC * _NS
    BPW = B // NW
    C = 64
    NCH = BPW // C
    IPC = C * F
    NSUB = IPC // 128
    assert B % NW == 0 and BPW % C == 0 and IPC % 128 == 0

    mesh = plsc.VectorSubcoreMesh(core_axis_name="c", subcore_axis_name="s")

    @functools.partial(
        pl.kernel,
        mesh=mesh,
        compiler_params=pltpu.CompilerParams(
            needs_layout_passes=False, use_tc_tiling_on_sc=False),
        out_type=jax.ShapeDtypeStruct((B,), jnp.float32),
        scratch_types=[
            pltpu.VMEM((IPC,), jnp.int32),
            pltpu.VMEM((NSUB, 128), jnp.int32),
            pltpu.VMEM((NSUB, 128), jnp.int32),
            pltpu.VMEM((IPC, _L), jnp.float32),
            pltpu.VMEM((IPC + _L,), jnp.float32),
            pltpu.VMEM((C,), jnp.float32),
            pltpu.VMEM((_L,), jnp.float32),
            pltpu.SemaphoreType.DMA,
        ],
    )
    def fm(x_hbm, emb_hbm, lin_hbm, lb_hbm, out_hbm,
           x_v, idx_v, slot_v, rows_v, lin_v, out_v, lb_v, sem):
        cid = lax.axis_index("c")
        sid = lax.axis_index("s")
        wid = sid * _NC + cid
        base = wid * BPW

        pltpu.sync_copy(lb_hbm, lb_v.at[pl.ds(0, 1)])
        lb = lb_v[pl.ds(0, _L)][0]

        lane = lax.iota(jnp.int32, _L)
        m_tail = jnp.where(lane < (F - _L), 1.0, 0.0).astype(jnp.float32)
        m_last = lane == (_L - 1)
        zeros_i = jnp.zeros((_L,), jnp.int32)

        def chunk_body(ci, carry):
            cbase = base + ci * C
            pltpu.sync_copy(x_hbm.at[pl.ds(cbase * F, IPC)], x_v)

            def idx_body(r, carry2):
                for j in range(128 // _L):
                    p0 = r * 128 + j * _L
                    pos = p0 + lane
                    fld = lax.rem(pos, F)
                    idx = x_v[pl.ds(p0, _L)] + fld * V
                    idx_v[r, pl.ds(j * _L, _L)] = idx
                    slot_v[r, pl.ds(j * _L, _L)] = (
                        (idx & jnp.int32(~131071))
                        | ((idx & 16383) << 3)
                        | ((idx >> 14) & 7))
                return carry2
            lax.fori_loop(0, NSUB, idx_body, 0)

            copies = []
            for j in range(NSUB):
                copies.append(pltpu.async_copy(
                    emb_hbm.at[slot_v.at[j]],
                    rows_v.at[pl.ds(j * 128, 128)], sem))
                copies.append(pltpu.async_copy(
                    lin_hbm.at[idx_v.at[j]],
                    lin_v.at[pl.ds(j * 128, 128)], sem))
            for cp in copies:
                cp.wait()

            def row_body(b, carry2):
                rbase = b * F
                acc = rows_v[rbase, :]
                acc2 = acc * acc
                for f in range(1, F):
                    r = rows_v[rbase + f, :]
                    acc = acc + r
                    acc2 = acc2 + r * r
                lv = lin_v[pl.ds(rbase, _L)] + lin_v[pl.ds(rbase + _L, _L)] * m_tail
                cs = plsc.cumsum(acc)
                cq = plsc.cumsum(acc2)
                cl = plsc.cumsum(lv)
                logit = cl + lb + 0.5 * (cs * cs - cq)
                sig = 1.0 / (1.0 + jnp.exp(-logit))
                plsc.store_scatter(out_v, [zeros_i + b], sig, mask=m_last)
                return carry2
            lax.fori_loop(0, C, row_body, 0)

            pltpu.sync_copy(out_v, out_hbm.at[pl.ds(cbase, C)])
            return carry
        lax.fori_loop(0, NCH, chunk_body, 0)

    return fm


def _build_transpose(FV, D):
    NW = _NC * _NS
    W = 128
    J = 8
    CW = J * W
    NTC = FV // W
    TAIL = FV - NTC * W
    NB = NTC // J
    KMAX = (NB + NW - 1) // NW
    assert NTC % J == 0 and KMAX % 2 == 0 and TAIL * D % W == 0

    mesh = plsc.VectorSubcoreMesh(core_axis_name="c", subcore_axis_name="s")

    @functools.partial(
        pl.kernel,
        mesh=mesh,
        compiler_params=pltpu.CompilerParams(
            needs_layout_passes=False, use_tc_tiling_on_sc=True),
        out_type=jax.ShapeDtypeStruct((FV * D // W, W), jnp.float32),
        scratch_types=[
            pltpu.VMEM((2, D, CW + 1), jnp.float32),
            pltpu.VMEM((2, CW * D // W, W), jnp.float32),
            pltpu.VMEM((TAIL * D // W, W), jnp.float32),
            pltpu.SemaphoreType.DMA,
            pltpu.SemaphoreType.DMA,
            pltpu.SemaphoreType.DMA,
            pltpu.SemaphoreType.DMA,
        ],
    )
    def transpose(src_hbm, tail_hbm, dst_hbm, tv, rv, lv, si0, si1, so0, so1):
        cid = lax.axis_index("c")
        sid = lax.axis_index("s")
        wid = sid * _NC + cid
        sin = (si0, si1)
        sout = (so0, so1)
        lane = lax.iota(jnp.int32, _L)
        RPB = CW * D // W

        def fire_in(m, p):
            @pl.when(m < NB)
            def _():
                pltpu.async_copy(
                    src_hbm.at[:, pl.ds(m * CW, CW)],
                    tv.at[p].at[:, pl.ds(0, CW)], sin[p])

        def wait_in(m, p):
            @pl.when(m < NB)
            def _():
                pltpu.make_async_copy(
                    src_hbm.at[:, pl.ds(m * CW, CW)],
                    tv.at[p].at[:, pl.ds(0, CW)], sin[p]).wait()

        def fire_out(m, p):
            @pl.when(m < NB)
            def _():
                pltpu.async_copy(
                    rv.at[p], dst_hbm.at[pl.ds(m * RPB, RPB)], sout[p])

        def wait_out(m, p):
            @pl.when(m < NB)
            def _():
                pltpu.make_async_copy(
                    rv.at[p], dst_hbm.at[pl.ds(m * RPB, RPB)],
                    sout[p]).wait()

        def trans(m, p):
            @pl.when(m < NB)
            def _():
                def step(i, carry):
                    for k in range(J):
                        r = i * J + k
                        vals = plsc.load_gather(tv.at[p], [lane, r + 0 * lane])
                        rv[p, i, pl.ds(k * D, D)] = vals
                    return carry
                lax.fori_loop(0, RPB, step, 0)

        def blk(k):
            return wid + NW * k

        fire_in(blk(0), 0)

        def pair_body(g, carry):
            k0 = 2 * g
            fire_in(blk(k0 + 1), 1)
            wait_in(blk(k0), 0)

            @pl.when(g > 0)
            def _():
                wait_out(blk(k0 - 2), 0)
            trans(blk(k0), 0)
            fire_out(blk(k0), 0)

            fire_in(blk(k0 + 2), 0)
            wait_in(blk(k0 + 1), 1)

            @pl.when(g > 0)
            def _():
                wait_out(blk(k0 - 1), 1)
            trans(blk(k0 + 1), 1)
            fire_out(blk(k0 + 1), 1)
            return carry
        lax.fori_loop(0, KMAX // 2, pair_body, 0)

        wait_in(blk(KMAX), 0)
        wait_out(blk(KMAX - 2), 0)
        wait_out(blk(KMAX - 1), 1)

        @pl.when(wid == NW - 1)
        def _():
            pltpu.sync_copy(tail_hbm, lv)
            pltpu.sync_copy(
                lv, dst_hbm.at[pl.ds(NTC * W * D // W, TAIL * D // W)])

    return transpose


def _build_tc_transpose(FV, D):
    BN = 131072
    NBLK = (FV + BN - 1) // BN
    SB = BN // 8
    assert BN % 128 == 0

    def body(x_ref, o_ref):
        eye_d = jnp.eye(D, dtype=jnp.float32)
        perm = jnp.eye(8 * D, dtype=jnp.float32).reshape(8, D, 8 * D)
        z = lax.dot_general(
            x_ref[...], eye_d, (((0,), (0,)), ((), ())),
            preferred_element_type=jnp.float32)
        out = None
        for rr in range(8):
            part = lax.dot_general(
                z[rr * SB:(rr + 1) * SB, :], perm[rr],
                (((1,), (0,)), ((), ())),
                preferred_element_type=jnp.float32)
            out = part if out is None else out + part
        o_ref[...] = out

    return pl.pallas_call(
        body,
        grid=(NBLK,),
        in_specs=[pl.BlockSpec((D, BN), lambda i: (0, i))],
        out_specs=pl.BlockSpec((SB, 8 * D), lambda i: (i, 0)),
        out_shape=jax.ShapeDtypeStruct((NBLK * SB, 8 * D), jnp.float32),
    )


def kernel(x, emb_table, lin_w, lin_b):
    B, F = x.shape
    D = emb_table.shape[1]
    V = emb_table.shape[0] // F
    FV = F * V
    w2 = _build_tc_transpose(FV, D)(emb_table.T)
    emb_2d = w2.reshape(w2.shape[0] * w2.shape[1] // D, D)
    fm = _build(B, F, V, D)
    out = fm(x.reshape(B * F).astype(jnp.int32), emb_2d, lin_w, lin_b)
    return out.reshape(B, 1)
